# Initial kernel scaffold; baseline (speedup 1.0000x reference)
import jax
import jax.numpy as jnp
from jax.experimental import pallas as pl


def kernel(device_embeddings, candidate_embedding, edge_index, Wl, bl, Wr, br, att, Wres, conv_bias, ln_gamma, ln_beta, Wout, bout):
    raise NotImplementedError("write your pallas kernel here")


# speedup vs baseline: 4.5343x; 4.5343x over previous
import functools

import jax
import jax.numpy as jnp
from jax.experimental import pallas as pl
from jax.experimental.pallas import tpu as pltpu

N_DEV = 10000
N_CAND = 10000
E = 160000
D = 256
H = 4
C = 256
HC = H * C

NPAD = 10240
BR = 512


def _leaky(x, slope):
    return jnp.where(x >= 0, x, slope * x)


def _proj_body(dev_ref, cand_ref, Wl_ref, bl_ref, Wr_ref, br_ref, Wres_ref,
               xl_ref, xr_ref, res_ref):
    xl_ref[...] = (
        jnp.dot(dev_ref[...], Wl_ref[...], preferred_element_type=jnp.float32)
        + bl_ref[...]
    )
    xr_ref[...] = (
        jnp.dot(cand_ref[...], Wr_ref[...], preferred_element_type=jnp.float32)
        + br_ref[...]
    )
    res_ref[...] = jnp.dot(cand_ref[...], Wres_ref[...],
                           preferred_element_type=jnp.float32)


def _proj(dev, cand, Wl, bl, Wr, br, Wres):
    grid = (NPAD // BR,)
    return pl.pallas_call(
        _proj_body,
        grid=grid,
        in_specs=[
            pl.BlockSpec((BR, D), lambda i: (i, 0)),
            pl.BlockSpec((BR, D), lambda i: (i, 0)),
            pl.BlockSpec((D, HC), lambda i: (0, 0)),
            pl.BlockSpec((1, HC), lambda i: (0, 0)),
            pl.BlockSpec((D, HC), lambda i: (0, 0)),
            pl.BlockSpec((1, HC), lambda i: (0, 0)),
            pl.BlockSpec((D, C), lambda i: (0, 0)),
        ],
        out_specs=[
            pl.BlockSpec((BR, HC), lambda i: (i, 0)),
            pl.BlockSpec((BR, HC), lambda i: (i, 0)),
            pl.BlockSpec((BR, C), lambda i: (i, 0)),
        ],
        out_shape=[
            jax.ShapeDtypeStruct((NPAD, HC), jnp.float32),
            jax.ShapeDtypeStruct((NPAD, HC), jnp.float32),
            jax.ShapeDtypeStruct((NPAD, C), jnp.float32),
        ],
    )(dev, cand, Wl, bl, Wr, br, Wres)


def _post_body(acc_ref, res_ref, cb_ref, g_ref, b_ref, Wout_ref, bout_ref,
               out_ref):
    x = acc_ref[...] + res_ref[...] + cb_ref[...]
    mu = jnp.mean(x, axis=-1, keepdims=True)
    var = jnp.mean((x - mu) ** 2, axis=-1, keepdims=True)
    xn = (x - mu) * jax.lax.rsqrt(var + 1e-5) * g_ref[...] + b_ref[...]
    act = _leaky(xn, 0.01)
    out_ref[...] = (
        jnp.dot(act, Wout_ref[...], preferred_element_type=jnp.float32)
        + bout_ref[...]
    )


def _post(acc, res, conv_bias, ln_gamma, ln_beta, Wout, bout):
    grid = (NPAD // BR,)
    return pl.pallas_call(
        _post_body,
        grid=grid,
        in_specs=[
            pl.BlockSpec((BR, C), lambda i: (i, 0)),
            pl.BlockSpec((BR, C), lambda i: (i, 0)),
            pl.BlockSpec((1, C), lambda i: (0, 0)),
            pl.BlockSpec((1, C), lambda i: (0, 0)),
            pl.BlockSpec((1, C), lambda i: (0, 0)),
            pl.BlockSpec((C, C), lambda i: (0, 0)),
            pl.BlockSpec((1, C), lambda i: (0, 0)),
        ],
        out_specs=pl.BlockSpec((BR, C), lambda i: (i, 0)),
        out_shape=jax.ShapeDtypeStruct((NPAD, C), jnp.float32),
    )(acc, res, conv_bias, ln_gamma, ln_beta, Wout, bout)


def kernel(device_embeddings, candidate_embedding, edge_index, Wl, bl, Wr, br,
           att, Wres, conv_bias, ln_gamma, ln_beta, Wout, bout):
    dev_p = jnp.pad(device_embeddings, ((0, NPAD - N_DEV), (0, 0)))
    cand_p = jnp.pad(candidate_embedding, ((0, NPAD - N_CAND), (0, 0)))
    xl, xr, res = _proj(dev_p, cand_p, Wl, bl[None, :], Wr, br[None, :], Wres)

    src = edge_index[0]
    dst = edge_index[1]
    xl_h = xl[:N_DEV].reshape(N_DEV, H, C)
    xr_h = xr[:N_CAND].reshape(N_CAND, H, C)
    e = _leaky(xl_h[src] + xr_h[dst], 0.2)
    logits = (e * att[None, :, :]).sum(-1)
    ex = jnp.exp(logits)
    denom = jax.ops.segment_sum(ex, dst, num_segments=N_CAND)
    alpha = ex / (denom[dst] + 1e-16)
    agg = jax.ops.segment_sum((alpha[:, :, None] * xl_h[src]).mean(axis=1),
                              dst, num_segments=N_CAND)
    acc = jnp.pad(agg, ((0, NPAD - N_CAND), (0, 0)))

    out = _post(acc, res, conv_bias[None, :], ln_gamma[None, :],
                ln_beta[None, :], Wout, bout[None, :])
    return out[:N_CAND]
